# baseline (speedup 1.0000x reference)
import numpy as np
import jax
import jax.numpy as jnp
from jax import lax
from jax.experimental import pallas as pl
from jax.experimental.pallas import tpu as pltpu
from jax.experimental.pallas import tpu_sc as plsc

H_UNITS_K = 128


def _pos_enc_table_np(seq, h_units):
    pos = np.arange(seq).astype(np.float64)[:, None]
    i = np.arange(h_units).astype(np.float64)[None, :]
    enc = pos / np.power(10000.0, 2.0 * i / float(h_units))
    enc = enc.astype(np.float32)
    enc[:, 0::2] = np.sin(enc[:, 0::2])
    enc[:, 1::2] = np.cos(enc[:, 1::2])
    return enc


def kernel(inputs):
    bs, seq = inputs.shape
    h = H_UNITS_K
    table = jnp.asarray(_pos_enc_table_np(seq, h))

    info = plsc.get_sparse_core_info()
    nc, ns = info.num_cores, info.num_subcores
    nw = nc * ns
    assert bs % nw == 0
    b_per_w = bs // nw
    mesh = plsc.VectorSubcoreMesh(core_axis_name="c", subcore_axis_name="s")

    rep = 4
    assert b_per_w % rep == 0

    def body(table_hbm, out_hbm, tab_v, sem):
        wid = lax.axis_index("s") * nc + lax.axis_index("c")
        for r in range(rep):
            pltpu.async_copy(table_hbm, tab_v.at[pl.ds(r * seq, seq)], sem)
        for r in range(rep):
            pltpu.make_async_copy(
                table_hbm, tab_v.at[pl.ds(r * seq, seq)], sem
            ).wait()
        base = wid * (b_per_w * seq)
        copies = []
        for b in range(0, b_per_w, rep):
            copies.append(
                pltpu.async_copy(
                    tab_v, out_hbm.at[pl.ds(base + b * seq, rep * seq)], sem
                )
            )
        for c in copies:
            c.wait()

    k = pl.kernel(
        body,
        mesh=mesh,
        out_type=jax.ShapeDtypeStruct((bs * seq, h), jnp.float32),
        scratch_types=[
            pltpu.VMEM((rep * seq, h), jnp.float32),
            pltpu.SemaphoreType.DMA,
        ],
    )
    out = k(table)
    return out.reshape(bs, seq, h)


# speedup vs baseline: 2.0937x; 2.0937x over previous
import numpy as np
import jax
import jax.numpy as jnp
from jax.experimental import pallas as pl

H_UNITS_K = 128


def _pos_enc_table_np(seq, h_units):
    pos = np.arange(seq).astype(np.float64)[:, None]
    i = np.arange(h_units).astype(np.float64)[None, :]
    enc = pos / np.power(10000.0, 2.0 * i / float(h_units))
    enc = enc.astype(np.float32)
    enc[:, 0::2] = np.sin(enc[:, 0::2])
    enc[:, 1::2] = np.cos(enc[:, 1::2])
    return enc


def kernel(inputs):
    bs, seq = inputs.shape
    h = H_UNITS_K
    table = jnp.asarray(_pos_enc_table_np(seq, h))

    BB = 32
    assert bs % BB == 0

    def body(tab_ref, out_ref):
        out_ref[...] = jnp.broadcast_to(tab_ref[...][None], (BB, seq, h))

    out = pl.pallas_call(
        body,
        grid=(bs // BB,),
        in_specs=[pl.BlockSpec((seq, h), lambda i: (0, 0))],
        out_specs=pl.BlockSpec((BB, seq, h), lambda i: (i, 0, 0)),
        out_shape=jax.ShapeDtypeStruct((bs, seq, h), jnp.float32),
    )(table)
    return out
